# Initial kernel scaffold; baseline (speedup 1.0000x reference)
import functools

import jax
import jax.numpy as jnp
from jax import lax
from jax.experimental import pallas as pl
from jax.experimental.pallas import tpu as pltpu
from jax.experimental.pallas import tpu_sc as plsc

N = 10000
E = 320000
D_IN = 128
HEADS = 8
HID = 16
HC1 = HEADS * HID
D_OUT = 64

_BLK = 1000


def _leaky_exp(z):
    return jnp.exp(jnp.where(z >= 0.0, z, z * 0.2))


def _lin1_body(x_ref, w1_ref, asr_ref, adr_ref, hp_ref, as_ref, ad_ref):
    h = jnp.dot(x_ref[...], w1_ref[...], preferred_element_type=jnp.float32)
    hp_ref[...] = h
    as_ref[...] = jnp.dot(h, asr_ref[...], preferred_element_type=jnp.float32)
    ad_ref[...] = jnp.dot(h, adr_ref[...], preferred_element_type=jnp.float32)


def _lin1(x, W1, Asrc1c, Adst1c):
    grid = (N // _BLK,)
    return pl.pallas_call(
        _lin1_body,
        grid=grid,
        in_specs=[
            pl.BlockSpec((_BLK, D_IN), lambda i: (i, 0)),
            pl.BlockSpec((D_IN, HC1), lambda i: (0, 0)),
            pl.BlockSpec((D_IN, 16), lambda i: (0, 0)),
            pl.BlockSpec((D_IN, 16), lambda i: (0, 0)),
        ],
        out_specs=[
            pl.BlockSpec((_BLK, HC1), lambda i: (i, 0)),
            pl.BlockSpec((_BLK, 16), lambda i: (i, 0)),
            pl.BlockSpec((_BLK, 16), lambda i: (i, 0)),
        ],
        out_shape=[
            jax.ShapeDtypeStruct((N, HC1), jnp.float32),
            jax.ShapeDtypeStruct((N, 16), jnp.float32),
            jax.ShapeDtypeStruct((N, 16), jnp.float32),
        ],
    )(x, W1, Asrc1c, Adst1c)


def _mid_body(p0_ref, p1_ref, h1_ref, asc_ref, adc_ref, b1_ref, w2_ref,
              me_ref, as2_ref, ad2_ref, hp2_ref, s2_ref, d2_ref):
    t = p0_ref[...] + p1_ref[...]
    w = _leaky_exp(asc_ref[...] + adc_ref[...])
    me = me_ref[...]
    wx = jnp.dot(w, me, preferred_element_type=jnp.float32)
    num = t[:, :HC1] + wx * h1_ref[...]
    den = jnp.dot(t[:, HC1:HC1 + 16] + w, me, preferred_element_type=jnp.float32)
    o = num / (den + 1e-16) + b1_ref[...]
    a = jnp.where(o > 0.0, o, jnp.expm1(o))
    h2 = jnp.dot(a, w2_ref[...], preferred_element_type=jnp.float32)
    hp2_ref[...] = h2
    s2_ref[...] = jnp.dot(h2, as2_ref[...], preferred_element_type=jnp.float32)
    d2_ref[...] = jnp.dot(h2, ad2_ref[...], preferred_element_type=jnp.float32)


def _mid(p0, p1, h1, as1c, ad1c, b1, W2, Me, As2c, Ad2c):
    R1 = HC1 + 16
    grid = (N // _BLK,)
    return pl.pallas_call(
        _mid_body,
        grid=grid,
        in_specs=[
            pl.BlockSpec((_BLK, R1), lambda i: (i, 0)),
            pl.BlockSpec((_BLK, R1), lambda i: (i, 0)),
            pl.BlockSpec((_BLK, HC1), lambda i: (i, 0)),
            pl.BlockSpec((_BLK, 16), lambda i: (i, 0)),
            pl.BlockSpec((_BLK, 16), lambda i: (i, 0)),
            pl.BlockSpec((1, HC1), lambda i: (0, 0)),
            pl.BlockSpec((HC1, D_OUT), lambda i: (0, 0)),
            pl.BlockSpec((16, HC1), lambda i: (0, 0)),
            pl.BlockSpec((D_OUT, 16), lambda i: (0, 0)),
            pl.BlockSpec((D_OUT, 16), lambda i: (0, 0)),
        ],
        out_specs=[
            pl.BlockSpec((_BLK, D_OUT), lambda i: (i, 0)),
            pl.BlockSpec((_BLK, 16), lambda i: (i, 0)),
            pl.BlockSpec((_BLK, 16), lambda i: (i, 0)),
        ],
        out_shape=[
            jax.ShapeDtypeStruct((N, D_OUT), jnp.float32),
            jax.ShapeDtypeStruct((N, 16), jnp.float32),
            jax.ShapeDtypeStruct((N, 16), jnp.float32),
        ],
    )(p0, p1, h1, as1c, ad1c, b1, W2, Me, As2c, Ad2c)


def _fin_body(p0_ref, p1_ref, h2_ref, asc_ref, adc_ref, b2_ref, m2_ref, o_ref):
    t = p0_ref[...] + p1_ref[...]
    w = _leaky_exp(asc_ref[...] + adc_ref[...])
    m2 = m2_ref[...]
    num = t[:, :D_OUT] + jnp.dot(w, m2, preferred_element_type=jnp.float32) * h2_ref[...]
    den = jnp.dot(t[:, D_OUT:D_OUT + 16] + w, m2, preferred_element_type=jnp.float32)
    o_ref[...] = num / (den + 1e-16) + b2_ref[...]


def _fin(p0, p1, h2, as2c, ad2c, b2, M2e):
    R2 = D_OUT + 16
    grid = (N // _BLK,)
    return pl.pallas_call(
        _fin_body,
        grid=grid,
        in_specs=[
            pl.BlockSpec((_BLK, R2), lambda i: (i, 0)),
            pl.BlockSpec((_BLK, R2), lambda i: (i, 0)),
            pl.BlockSpec((_BLK, D_OUT), lambda i: (i, 0)),
            pl.BlockSpec((_BLK, 16), lambda i: (i, 0)),
            pl.BlockSpec((_BLK, 16), lambda i: (i, 0)),
            pl.BlockSpec((1, D_OUT), lambda i: (0, 0)),
            pl.BlockSpec((16, D_OUT), lambda i: (0, 0)),
        ],
        out_specs=pl.BlockSpec((_BLK, D_OUT), lambda i: (i, 0)),
        out_shape=jax.ShapeDtypeStruct((N, D_OUT), jnp.float32),
    )(p0, p1, h2, as2c, ad2c, b2, M2e)


def _make_edge_kernel(hw: int, k: int):
    r = hw + 16
    nj = hw // 16
    epw = E // 32
    nch = epw // k
    rp = N // 16
    mesh = plsc.VectorSubcoreMesh(core_axis_name="c", subcore_axis_name="s")

    @functools.partial(
        pl.kernel,
        out_type=jax.ShapeDtypeStruct((2, N, r), jnp.float32),
        mesh=mesh,
        scratch_types=[
            pltpu.VMEM((k,), jnp.int32),
            pltpu.VMEM((k,), jnp.int32),
            pltpu.VMEM((k, hw), jnp.float32),
            pltpu.VMEM((k, 16), jnp.float32),
            pltpu.VMEM((k, 16), jnp.float32),
            pltpu.VMEM((k, r), jnp.float32),
            pltpu.VMEM_SHARED((N, r), jnp.float32),
            pltpu.SemaphoreType.DMA,
            pltpu.SemaphoreType.DMA,
            pltpu.SemaphoreType.DMA,
        ],
    )
    def ek(src_h, dst_h, hp_h, as_h, ad_h, zz_h, out_h,
           sidx, didx, hpb, asb, adb, mb, acc, s1, s2, s3):
        c = lax.axis_index("c")
        s = lax.axis_index("s")
        wid = c * 16 + s
        roff = s * rp
        pltpu.sync_copy(zz_h.at[pl.ds(roff, rp)], acc.at[pl.ds(roff, rp)])
        plsc.subcore_barrier()
        base = wid * epw

        def chunk(i, carry):
            off = base + i * k
            pltpu.sync_copy(src_h.at[pl.ds(off, k)], sidx)
            pltpu.sync_copy(dst_h.at[pl.ds(off, k)], didx)
            d1 = pltpu.async_copy(hp_h.at[sidx], hpb, s1)
            d2 = pltpu.async_copy(as_h.at[sidx], asb, s2)
            d3 = pltpu.async_copy(ad_h.at[didx], adb, s3)
            d1.wait()
            d2.wait()
            d3.wait()

            def edge(e, cc):
                z = asb[e, :] + adb[e, :]
                w = jnp.exp(jnp.where(z >= 0.0, z, z * 0.2))
                mb[e, pl.ds(hw, 16)] = w
                for j in range(nj):
                    ws = mb[e, hw + j]
                    mb[e, pl.ds(j * 16, 16)] = (
                        hpb[e, pl.ds(j * 16, 16)] * jnp.full((16,), ws, jnp.float32))
                return cc

            lax.fori_loop(0, k, edge, 0)
            pltpu.sync_copy(mb, acc.at[didx], add=True)
            return carry

        lax.fori_loop(0, nch, chunk, 0)
        plsc.subcore_barrier()
        pltpu.sync_copy(acc.at[pl.ds(roff, rp)], out_h.at[c, pl.ds(roff, rp)])

    return ek


_edge1 = _make_edge_kernel(HC1, 80)
_edge2 = _make_edge_kernel(D_OUT, 80)


def kernel(x, edge_index, W1, att_src1, att_dst1, bias1,
           W2, att_src2, att_dst2, bias2):
    src = edge_index[0]
    dst = edge_index[1]

    asf1 = att_src1.reshape(HC1)
    adf1 = att_dst1.reshape(HC1)
    i128 = jnp.arange(HC1)
    h16 = jnp.arange(16)
    Mc = ((i128[:, None] // HID) == h16[None, :]).astype(jnp.float32)
    Asrc1c = asf1[:, None] * Mc
    Adst1c = adf1[:, None] * Mc
    Me = (h16[:, None] == (i128[None, :] // HID)).astype(jnp.float32)
    As2c = jnp.concatenate([att_src2.reshape(D_OUT, 1),
                            jnp.zeros((D_OUT, 15), jnp.float32)], axis=1)
    Ad2c = jnp.concatenate([att_dst2.reshape(D_OUT, 1),
                            jnp.zeros((D_OUT, 15), jnp.float32)], axis=1)
    M2e = jnp.zeros((16, D_OUT), jnp.float32).at[0].set(1.0)

    h1, as1c, ad1c = _lin1(x, W1, Asrc1c, Adst1c)

    z1 = jnp.zeros((N, HC1 + 16), jnp.float32)
    part1 = _edge1(src, dst, h1, as1c, ad1c, z1)

    h2, as2c, ad2c = _mid(part1[0], part1[1], h1, as1c, ad1c,
                          bias1.reshape(1, HC1), W2, Me, As2c, Ad2c)

    z2 = jnp.zeros((N, D_OUT + 16), jnp.float32)
    part2 = _edge2(src, dst, h2, as2c, ad2c, z2)

    out = _fin(part2[0], part2[1], h2, as2c, ad2c,
               bias2.reshape(1, D_OUT), M2e)
    return out


# speedup vs baseline: 34.5835x; 34.5835x over previous
import functools

import jax
import jax.numpy as jnp
from jax import lax
from jax.experimental import pallas as pl
from jax.experimental.pallas import tpu as pltpu
from jax.experimental.pallas import tpu_sc as plsc

N = 10000
E = 320000
D_IN = 128
HEADS = 8
HID = 16
HC1 = HEADS * HID
D_OUT = 64

_BLK = 1000


def _leaky_exp(z):
    return jnp.exp(jnp.where(z >= 0.0, z, z * 0.2))


def _lin1_body(x_ref, w1_ref, asr_ref, adr_ref, hp_ref, as_ref, ad_ref):
    h = jnp.dot(x_ref[...], w1_ref[...], preferred_element_type=jnp.float32)
    hp_ref[...] = h
    as_ref[...] = jnp.dot(h, asr_ref[...], preferred_element_type=jnp.float32)
    ad_ref[...] = jnp.dot(h, adr_ref[...], preferred_element_type=jnp.float32)


def _lin1(x, W1, Asrc1c, Adst1c):
    grid = (N // _BLK,)
    return pl.pallas_call(
        _lin1_body,
        grid=grid,
        in_specs=[
            pl.BlockSpec((_BLK, D_IN), lambda i: (i, 0)),
            pl.BlockSpec((D_IN, HC1), lambda i: (0, 0)),
            pl.BlockSpec((D_IN, 16), lambda i: (0, 0)),
            pl.BlockSpec((D_IN, 16), lambda i: (0, 0)),
        ],
        out_specs=[
            pl.BlockSpec((_BLK, HC1), lambda i: (i, 0)),
            pl.BlockSpec((_BLK, 16), lambda i: (i, 0)),
            pl.BlockSpec((_BLK, 16), lambda i: (i, 0)),
        ],
        out_shape=[
            jax.ShapeDtypeStruct((N, HC1), jnp.float32),
            jax.ShapeDtypeStruct((N, 16), jnp.float32),
            jax.ShapeDtypeStruct((N, 16), jnp.float32),
        ],
    )(x, W1, Asrc1c, Adst1c)


def _mid_body(p0_ref, p1_ref, h1_ref, asc_ref, adc_ref, b1_ref, w2_ref,
              me_ref, as2_ref, ad2_ref, hp2_ref, s2_ref, d2_ref):
    t = p0_ref[...] + p1_ref[...]
    w = _leaky_exp(asc_ref[...] + adc_ref[...])
    me = me_ref[...]
    wx = jnp.dot(w, me, preferred_element_type=jnp.float32)
    num = t[:, :HC1] + wx * h1_ref[...]
    den = jnp.dot(t[:, HC1:HC1 + 16] + w, me, preferred_element_type=jnp.float32)
    o = num / (den + 1e-16) + b1_ref[...]
    a = jnp.where(o > 0.0, o, jnp.exp(o) - 1.0)
    h2 = jnp.dot(a, w2_ref[...], preferred_element_type=jnp.float32)
    hp2_ref[...] = h2
    s2_ref[...] = jnp.dot(h2, as2_ref[...], preferred_element_type=jnp.float32)
    d2_ref[...] = jnp.dot(h2, ad2_ref[...], preferred_element_type=jnp.float32)


def _mid(p0, p1, h1, as1c, ad1c, b1, W2, Me, As2c, Ad2c):
    R1 = HC1 + 16
    grid = (N // _BLK,)
    return pl.pallas_call(
        _mid_body,
        grid=grid,
        in_specs=[
            pl.BlockSpec((_BLK, R1), lambda i: (i, 0)),
            pl.BlockSpec((_BLK, R1), lambda i: (i, 0)),
            pl.BlockSpec((_BLK, HC1), lambda i: (i, 0)),
            pl.BlockSpec((_BLK, 16), lambda i: (i, 0)),
            pl.BlockSpec((_BLK, 16), lambda i: (i, 0)),
            pl.BlockSpec((1, HC1), lambda i: (0, 0)),
            pl.BlockSpec((HC1, D_OUT), lambda i: (0, 0)),
            pl.BlockSpec((16, HC1), lambda i: (0, 0)),
            pl.BlockSpec((D_OUT, 16), lambda i: (0, 0)),
            pl.BlockSpec((D_OUT, 16), lambda i: (0, 0)),
        ],
        out_specs=[
            pl.BlockSpec((_BLK, D_OUT), lambda i: (i, 0)),
            pl.BlockSpec((_BLK, 16), lambda i: (i, 0)),
            pl.BlockSpec((_BLK, 16), lambda i: (i, 0)),
        ],
        out_shape=[
            jax.ShapeDtypeStruct((N, D_OUT), jnp.float32),
            jax.ShapeDtypeStruct((N, 16), jnp.float32),
            jax.ShapeDtypeStruct((N, 16), jnp.float32),
        ],
    )(p0, p1, h1, as1c, ad1c, b1, W2, Me, As2c, Ad2c)


def _fin_body(p0_ref, p1_ref, h2_ref, asc_ref, adc_ref, b2_ref, m2_ref, o_ref):
    t = p0_ref[...] + p1_ref[...]
    w = _leaky_exp(asc_ref[...] + adc_ref[...])
    m2 = m2_ref[...]
    num = t[:, :D_OUT] + jnp.dot(w, m2, preferred_element_type=jnp.float32) * h2_ref[...]
    den = jnp.dot(t[:, D_OUT:D_OUT + 16] + w, m2, preferred_element_type=jnp.float32)
    o_ref[...] = num / (den + 1e-16) + b2_ref[...]


def _fin(p0, p1, h2, as2c, ad2c, b2, M2e):
    R2 = D_OUT + 16
    grid = (N // _BLK,)
    return pl.pallas_call(
        _fin_body,
        grid=grid,
        in_specs=[
            pl.BlockSpec((_BLK, R2), lambda i: (i, 0)),
            pl.BlockSpec((_BLK, R2), lambda i: (i, 0)),
            pl.BlockSpec((_BLK, D_OUT), lambda i: (i, 0)),
            pl.BlockSpec((_BLK, 16), lambda i: (i, 0)),
            pl.BlockSpec((_BLK, 16), lambda i: (i, 0)),
            pl.BlockSpec((1, D_OUT), lambda i: (0, 0)),
            pl.BlockSpec((16, D_OUT), lambda i: (0, 0)),
        ],
        out_specs=pl.BlockSpec((_BLK, D_OUT), lambda i: (i, 0)),
        out_shape=jax.ShapeDtypeStruct((N, D_OUT), jnp.float32),
    )(p0, p1, h2, as2c, ad2c, b2, M2e)


def _make_edge_kernel(hw: int, k: int):
    r = hw + 16
    nj = hw // 16
    epw = E // 32
    nch = epw // k
    rp = (N // 16) // 8 * 8
    tail = N - 16 * rp
    mesh = plsc.VectorSubcoreMesh(core_axis_name="c", subcore_axis_name="s")

    @functools.partial(
        pl.kernel,
        out_type=jax.ShapeDtypeStruct((2, N, r), jnp.float32),
        mesh=mesh,
        compiler_params=pltpu.CompilerParams(use_tc_tiling_on_sc=False),
        scratch_types=[
            pltpu.VMEM((k,), jnp.int32),
            pltpu.VMEM((k,), jnp.int32),
            pltpu.VMEM((k, hw), jnp.float32),
            pltpu.VMEM((k, 16), jnp.float32),
            pltpu.VMEM((k, 16), jnp.float32),
            pltpu.VMEM((k, r), jnp.float32),
            pltpu.VMEM_SHARED((N, r), jnp.float32),
            pltpu.SemaphoreType.DMA,
            pltpu.SemaphoreType.DMA,
            pltpu.SemaphoreType.DMA,
        ],
    )
    def ek(src_h, dst_h, hp_h, as_h, ad_h, zz_h, out_h,
           sidx, didx, hpb, asb, adb, mb, acc, s1, s2, s3):
        c = lax.axis_index("c")
        s = lax.axis_index("s")
        wid = c * 16 + s
        roff = s * rp
        pltpu.sync_copy(zz_h.at[pl.ds(roff, rp)], acc.at[pl.ds(roff, rp)])

        @pl.when(s == 0)
        def _():
            pltpu.sync_copy(zz_h.at[pl.ds(16 * rp, tail)],
                            acc.at[pl.ds(16 * rp, tail)])

        plsc.subcore_barrier()
        base = wid * epw

        def chunk(i, carry):
            off = base + i * k
            pltpu.sync_copy(src_h.at[pl.ds(off, k)], sidx)
            pltpu.sync_copy(dst_h.at[pl.ds(off, k)], didx)
            d1 = pltpu.async_copy(hp_h.at[sidx], hpb, s1)
            d2 = pltpu.async_copy(as_h.at[sidx], asb, s2)
            d3 = pltpu.async_copy(ad_h.at[didx], adb, s3)
            d1.wait()
            d2.wait()
            d3.wait()

            def edge(e, cc):
                z = asb[e, :] + adb[e, :]
                w = jnp.exp(jnp.where(z >= 0.0, z, z * 0.2))
                mb[e, pl.ds(hw, 16)] = w
                for j in range(nj):
                    ws = w[j]
                    mb[e, pl.ds(j * 16, 16)] = (
                        hpb[e, pl.ds(j * 16, 16)] * jnp.full((16,), ws, jnp.float32))
                return cc

            lax.fori_loop(0, k, edge, 0)
            pltpu.sync_copy(mb, acc.at[didx], add=True)
            return carry

        lax.fori_loop(0, nch, chunk, 0)
        plsc.subcore_barrier()
        pltpu.sync_copy(acc.at[pl.ds(roff, rp)], out_h.at[c, pl.ds(roff, rp)])

        @pl.when(s == 0)
        def _():
            pltpu.sync_copy(acc.at[pl.ds(16 * rp, tail)],
                            out_h.at[c, pl.ds(16 * rp, tail)])

    return ek


_edge1 = _make_edge_kernel(HC1, 80)
_edge2 = _make_edge_kernel(D_OUT, 80)


def kernel(x, edge_index, W1, att_src1, att_dst1, bias1,
           W2, att_src2, att_dst2, bias2):
    src = edge_index[0]
    dst = edge_index[1]

    asf1 = att_src1.reshape(HC1)
    adf1 = att_dst1.reshape(HC1)
    i128 = jnp.arange(HC1)
    h16 = jnp.arange(16)
    Mc = ((i128[:, None] // HID) == h16[None, :]).astype(jnp.float32)
    Asrc1c = asf1[:, None] * Mc
    Adst1c = adf1[:, None] * Mc
    Me = (h16[:, None] == (i128[None, :] // HID)).astype(jnp.float32)
    p2 = jnp.concatenate([jnp.ones((1, 4), jnp.float32),
                          jnp.zeros((1, 12), jnp.float32)], axis=1)
    As2c = att_src2.reshape(D_OUT, 1) @ p2
    Ad2c = att_dst2.reshape(D_OUT, 1) @ p2
    M2e = jnp.zeros((16, D_OUT), jnp.float32).at[0].set(1.0)

    h1, as1c, ad1c = _lin1(x, W1, Asrc1c, Adst1c)

    z1 = jnp.zeros((N, HC1 + 16), jnp.float32)
    part1 = _edge1(src, dst, h1, as1c, ad1c, z1)

    h2, as2c, ad2c = _mid(part1[0], part1[1], h1, as1c, ad1c,
                          bias1.reshape(1, HC1), W2, Me, As2c, Ad2c)

    z2 = jnp.zeros((N, D_OUT + 16), jnp.float32)
    part2 = _edge2(src, dst, h2, as2c, ad2c, z2)

    out = _fin(part2[0], part2[1], h2, as2c, ad2c,
               bias2.reshape(1, D_OUT), M2e)
    return out


# speedup vs baseline: 80.0995x; 2.3161x over previous
import functools

import jax
import jax.numpy as jnp
from jax import lax
from jax.experimental import pallas as pl
from jax.experimental.pallas import tpu as pltpu
from jax.experimental.pallas import tpu_sc as plsc

N = 10000
E = 320000
D_IN = 128
HEADS = 8
HID = 16
HC1 = HEADS * HID
D_OUT = 64

_BLK = 1000
_K = 80


def _leaky_exp(z):
    return jnp.exp(jnp.where(z >= 0.0, z, z * 0.2))


def _lin1_body(x_ref, w1_ref, cat_ref, adr_ref, hs_ref, ad_ref):
    h = jnp.dot(x_ref[...], w1_ref[...], preferred_element_type=jnp.float32)
    hs_ref[...] = jnp.dot(h, cat_ref[...], preferred_element_type=jnp.float32)
    ad_ref[...] = jnp.dot(h, adr_ref[...], preferred_element_type=jnp.float32)


def _lin1(x, W1, Cat1, Adst1c):
    R1 = HC1 + 16
    grid = (N // _BLK,)
    return pl.pallas_call(
        _lin1_body,
        grid=grid,
        in_specs=[
            pl.BlockSpec((_BLK, D_IN), lambda i: (i, 0)),
            pl.BlockSpec((D_IN, HC1), lambda i: (0, 0)),
            pl.BlockSpec((HC1, R1), lambda i: (0, 0)),
            pl.BlockSpec((HC1, 16), lambda i: (0, 0)),
        ],
        out_specs=[
            pl.BlockSpec((_BLK, R1), lambda i: (i, 0)),
            pl.BlockSpec((_BLK, 16), lambda i: (i, 0)),
        ],
        out_shape=[
            jax.ShapeDtypeStruct((N, R1), jnp.float32),
            jax.ShapeDtypeStruct((N, 16), jnp.float32),
        ],
    )(x, W1, Cat1, Adst1c)


def _mid_body(p0_ref, p1_ref, hs_ref, adc_ref, b1_ref, w2_ref,
              me_ref, cat2_ref, ad2_ref, hs2_ref, d2_ref):
    t = p0_ref[...] + p1_ref[...]
    h1 = hs_ref[:, :HC1]
    w = _leaky_exp(hs_ref[:, HC1:HC1 + 16] + adc_ref[...])
    me = me_ref[...]
    wx = jnp.dot(w, me, preferred_element_type=jnp.float32)
    num = t[:, :HC1] + wx * h1
    den = jnp.dot(t[:, HC1:HC1 + 16] + w, me, preferred_element_type=jnp.float32)
    o = num / (den + 1e-16) + b1_ref[...]
    a = jnp.where(o > 0.0, o, jnp.exp(o) - 1.0)
    h2 = jnp.dot(a, w2_ref[...], preferred_element_type=jnp.float32)
    hs2_ref[...] = jnp.dot(h2, cat2_ref[...], preferred_element_type=jnp.float32)
    d2_ref[...] = jnp.dot(h2, ad2_ref[...], preferred_element_type=jnp.float32)


def _mid(p0, p1, hs1, ad1c, b1, W2, Me, Cat2, Ad2c):
    R1 = HC1 + 16
    R2 = D_OUT + 16
    grid = (N // _BLK,)
    return pl.pallas_call(
        _mid_body,
        grid=grid,
        in_specs=[
            pl.BlockSpec((_BLK, R1), lambda i: (i, 0)),
            pl.BlockSpec((_BLK, R1), lambda i: (i, 0)),
            pl.BlockSpec((_BLK, R1), lambda i: (i, 0)),
            pl.BlockSpec((_BLK, 16), lambda i: (i, 0)),
            pl.BlockSpec((1, HC1), lambda i: (0, 0)),
            pl.BlockSpec((HC1, D_OUT), lambda i: (0, 0)),
            pl.BlockSpec((16, HC1), lambda i: (0, 0)),
            pl.BlockSpec((D_OUT, R2), lambda i: (0, 0)),
            pl.BlockSpec((D_OUT, 16), lambda i: (0, 0)),
        ],
        out_specs=[
            pl.BlockSpec((_BLK, R2), lambda i: (i, 0)),
            pl.BlockSpec((_BLK, 16), lambda i: (i, 0)),
        ],
        out_shape=[
            jax.ShapeDtypeStruct((N, R2), jnp.float32),
            jax.ShapeDtypeStruct((N, 16), jnp.float32),
        ],
    )(p0, p1, hs1, ad1c, b1, W2, Me, Cat2, Ad2c)


def _fin_body(p0_ref, p1_ref, hs2_ref, adc_ref, b2_ref, m2_ref, o_ref):
    t = p0_ref[...] + p1_ref[...]
    h2 = hs2_ref[:, :D_OUT]
    w = _leaky_exp(hs2_ref[:, D_OUT:D_OUT + 16] + adc_ref[...])
    m2 = m2_ref[...]
    num = t[:, :D_OUT] + jnp.dot(w, m2, preferred_element_type=jnp.float32) * h2
    den = jnp.dot(t[:, D_OUT:D_OUT + 16] + w, m2, preferred_element_type=jnp.float32)
    o_ref[...] = num / (den + 1e-16) + b2_ref[...]


def _fin(p0, p1, hs2, ad2c, b2, M2e):
    R2 = D_OUT + 16
    grid = (N // _BLK,)
    return pl.pallas_call(
        _fin_body,
        grid=grid,
        in_specs=[
            pl.BlockSpec((_BLK, R2), lambda i: (i, 0)),
            pl.BlockSpec((_BLK, R2), lambda i: (i, 0)),
            pl.BlockSpec((_BLK, R2), lambda i: (i, 0)),
            pl.BlockSpec((_BLK, 16), lambda i: (i, 0)),
            pl.BlockSpec((1, D_OUT), lambda i: (0, 0)),
            pl.BlockSpec((16, D_OUT), lambda i: (0, 0)),
        ],
        out_specs=pl.BlockSpec((_BLK, D_OUT), lambda i: (i, 0)),
        out_shape=jax.ShapeDtypeStruct((N, D_OUT), jnp.float32),
    )(p0, p1, hs2, ad2c, b2, M2e)


def _make_edge_kernel(hw: int, k: int):
    r = hw + 16
    nj = hw // 16
    epw = E // 32
    nch = epw // k
    assert nch % 2 == 1 and nch >= 3
    npair = (nch - 1) // 2
    rp = (N // 16) // 8 * 8
    tail = N - 16 * rp
    mesh = plsc.VectorSubcoreMesh(core_axis_name="c", subcore_axis_name="s")

    @functools.partial(
        pl.kernel,
        out_type=jax.ShapeDtypeStruct((2, N, r), jnp.float32),
        mesh=mesh,
        compiler_params=pltpu.CompilerParams(use_tc_tiling_on_sc=False),
        scratch_types=[
            pltpu.VMEM((k,), jnp.int32),
            pltpu.VMEM((k,), jnp.int32),
            pltpu.VMEM((k,), jnp.int32),
            pltpu.VMEM((k,), jnp.int32),
            pltpu.VMEM((k, r), jnp.float32),
            pltpu.VMEM((k, r), jnp.float32),
            pltpu.VMEM((k, 16), jnp.float32),
            pltpu.VMEM((k, 16), jnp.float32),
            pltpu.VMEM((k, r), jnp.float32),
            pltpu.VMEM_SHARED((N, r), jnp.float32),
            pltpu.SemaphoreType.DMA,
            pltpu.SemaphoreType.DMA,
            pltpu.SemaphoreType.DMA,
            pltpu.SemaphoreType.DMA,
            pltpu.SemaphoreType.DMA,
        ],
    )
    def ek(src_h, dst_h, hs_h, ad_h, zz_h, out_h,
           sidx0, sidx1, didx0, didx1, hsb0, hsb1, adb0, adb1, mb,
           acc, gi0, gi1, g0, g1, ss):
        c = lax.axis_index("c")
        s = lax.axis_index("s")
        wid = c * 16 + s
        roff = s * rp
        pltpu.sync_copy(zz_h.at[pl.ds(roff, rp)], acc.at[pl.ds(roff, rp)])

        @pl.when(s == 0)
        def _():
            pltpu.sync_copy(zz_h.at[pl.ds(16 * rp, tail)],
                            acc.at[pl.ds(16 * rp, tail)])

        plsc.subcore_barrier()
        base = wid * epw
        sb = (sidx0, sidx1)
        db = (didx0, didx1)
        hb = (hsb0, hsb1)
        ab = (adb0, adb1)
        gi = (gi0, gi1)
        gg = (g0, g1)

        def sid(i, p):
            pltpu.async_copy(src_h.at[pl.ds(base + i * k, k)], sb[p], gi[p])
            pltpu.async_copy(dst_h.at[pl.ds(base + i * k, k)], db[p], gi[p])

        def wid_(i, p):
            pltpu.make_async_copy(src_h.at[pl.ds(base + i * k, k)], sb[p], gi[p]).wait()
            pltpu.make_async_copy(dst_h.at[pl.ds(base + i * k, k)], db[p], gi[p]).wait()

        def sgath(p):
            pltpu.async_copy(hs_h.at[sb[p]], hb[p], gg[p])
            pltpu.async_copy(ad_h.at[db[p]], ab[p], gg[p])

        def wgath(p):
            pltpu.make_async_copy(hs_h.at[sb[p]], hb[p], gg[p]).wait()
            pltpu.make_async_copy(ad_h.at[db[p]], ab[p], gg[p]).wait()

        def comp(p):
            hsb = hb[p]
            adb = ab[p]

            def edge(e, cc):
                z = hsb[e, pl.ds(hw, 16)] + adb[e, :]
                w = jnp.exp(jnp.where(z >= 0.0, z, z * 0.2))
                mb[e, pl.ds(hw, 16)] = w
                for j in range(nj):
                    mb[e, pl.ds(j * 16, 16)] = (
                        hsb[e, pl.ds(j * 16, 16)] * jnp.full((16,), w[j], jnp.float32))
                return cc

            lax.fori_loop(0, k, edge, 0)

        def scat(p):
            pltpu.async_copy(mb, acc.at[db[p]], ss, add=True)

        def wscat(p):
            pltpu.make_async_copy(mb, acc.at[db[p]], ss).wait()

        sid(0, 0)
        wid_(0, 0)
        sgath(0)
        sid(1, 1)

        def half(cid, p, first, do_next, do_nextid):
            wgath(p)
            if do_next:
                wid_(cid + 1, 1 - p)
                sgath(1 - p)
            if not first:
                wscat(1 - p)
            comp(p)
            scat(p)
            if do_nextid:
                sid(cid + 2, p)

        def body(j, cc):
            a = 2 * j
            half(a, 0, False, True, True)

            @pl.when(j <= npair - 2)
            def _():
                half(a + 1, 1, False, True, True)

            @pl.when(j == npair - 1)
            def _():
                half(a + 1, 1, False, True, False)

            return cc

        half(0, 0, True, True, True)
        half(1, 1, False, True, True)
        lax.fori_loop(1, npair, body, 0)

        last = nch - 1
        half(last, 0, False, False, False)
        wscat(0)

        plsc.subcore_barrier()
        pltpu.sync_copy(acc.at[pl.ds(roff, rp)], out_h.at[c, pl.ds(roff, rp)])

        @pl.when(s == 0)
        def _():
            pltpu.sync_copy(acc.at[pl.ds(16 * rp, tail)],
                            out_h.at[c, pl.ds(16 * rp, tail)])

    return ek


_edge1 = _make_edge_kernel(HC1, _K)
_edge2 = _make_edge_kernel(D_OUT, _K)


def kernel(x, edge_index, W1, att_src1, att_dst1, bias1,
           W2, att_src2, att_dst2, bias2):
    src = edge_index[0]
    dst = edge_index[1]

    asf1 = att_src1.reshape(HC1)
    adf1 = att_dst1.reshape(HC1)
    i128 = jnp.arange(HC1)
    h16 = jnp.arange(16)
    Mc = ((i128[:, None] // HID) == h16[None, :]).astype(jnp.float32)
    Cat1 = jnp.concatenate([jnp.eye(HC1, dtype=jnp.float32),
                            asf1[:, None] * Mc], axis=1)
    Adst1c = adf1[:, None] * Mc
    Me = (h16[:, None] == (i128[None, :] // HID)).astype(jnp.float32)
    p2 = jnp.concatenate([jnp.ones((1, 4), jnp.float32),
                          jnp.zeros((1, 12), jnp.float32)], axis=1)
    Cat2 = jnp.concatenate([jnp.eye(D_OUT, dtype=jnp.float32),
                            att_src2.reshape(D_OUT, 1) @ p2], axis=1)
    Ad2c = att_dst2.reshape(D_OUT, 1) @ p2
    M2e = jnp.zeros((16, D_OUT), jnp.float32).at[0].set(1.0)

    hs1, ad1c = _lin1(x, W1, Cat1, Adst1c)

    z1 = jnp.zeros((N, HC1 + 16), jnp.float32)
    part1 = _edge1(src, dst, hs1, ad1c, z1)

    hs2, ad2c = _mid(part1[0], part1[1], hs1, ad1c,
                     bias1.reshape(1, HC1), W2, Me, Cat2, Ad2c)

    z2 = jnp.zeros((N, D_OUT + 16), jnp.float32)
    part2 = _edge2(src, dst, hs2, ad2c, z2)

    out = _fin(part2[0], part2[1], hs2, ad2c,
               bias2.reshape(1, D_OUT), M2e)
    return out


# speedup vs baseline: 113.6973x; 1.4195x over previous
import functools

import jax
import jax.numpy as jnp
from jax import lax
from jax.experimental import pallas as pl
from jax.experimental.pallas import tpu as pltpu
from jax.experimental.pallas import tpu_sc as plsc

N = 10000
E = 320000
D_IN = 128
HEADS = 8
HID = 16
HC1 = HEADS * HID
D_OUT = 64

_BLK = 1000
_K = 80


def _leaky_exp(z):
    return jnp.exp(jnp.where(z >= 0.0, z, z * 0.2))


def _lin1_body(x_ref, w1_ref, cat_ref, adr_ref, hs_ref, ad_ref):
    h = jnp.dot(x_ref[...], w1_ref[...], preferred_element_type=jnp.float32)
    hs_ref[...] = jnp.dot(h, cat_ref[...], preferred_element_type=jnp.float32)
    ad_ref[...] = jnp.dot(h, adr_ref[...], preferred_element_type=jnp.float32)


def _lin1(x, W1, Cat1, Adst1c):
    R1 = HC1 + 16
    grid = (N // _BLK,)
    return pl.pallas_call(
        _lin1_body,
        grid=grid,
        in_specs=[
            pl.BlockSpec((_BLK, D_IN), lambda i: (i, 0)),
            pl.BlockSpec((D_IN, HC1), lambda i: (0, 0)),
            pl.BlockSpec((HC1, R1), lambda i: (0, 0)),
            pl.BlockSpec((HC1, 16), lambda i: (0, 0)),
        ],
        out_specs=[
            pl.BlockSpec((_BLK, R1), lambda i: (i, 0)),
            pl.BlockSpec((_BLK, 16), lambda i: (i, 0)),
        ],
        out_shape=[
            jax.ShapeDtypeStruct((N, R1), jnp.float32),
            jax.ShapeDtypeStruct((N, 16), jnp.float32),
        ],
    )(x, W1, Cat1, Adst1c)


def _mid_body(p0_ref, p1_ref, hs_ref, adc_ref, b1_ref, w2_ref,
              me_ref, cat2_ref, ad2_ref, hs2_ref, d2_ref):
    t = p0_ref[...] + p1_ref[...]
    h1 = hs_ref[:, :HC1]
    w = _leaky_exp(hs_ref[:, HC1:HC1 + 16] + adc_ref[...])
    me = me_ref[...]
    wx = jnp.dot(w, me, preferred_element_type=jnp.float32)
    num = t[:, :HC1] + wx * h1
    den = jnp.dot(t[:, HC1:HC1 + 16] + w, me, preferred_element_type=jnp.float32)
    o = num / (den + 1e-16) + b1_ref[...]
    a = jnp.where(o > 0.0, o, jnp.exp(o) - 1.0)
    h2 = jnp.dot(a, w2_ref[...], preferred_element_type=jnp.float32)
    hs2_ref[...] = jnp.dot(h2, cat2_ref[...], preferred_element_type=jnp.float32)
    d2_ref[...] = jnp.dot(h2, ad2_ref[...], preferred_element_type=jnp.float32)


def _mid(p0, p1, hs1, ad1c, b1, W2, Me, Cat2, Ad2c):
    R1 = HC1 + 16
    R2 = D_OUT + 16
    grid = (N // _BLK,)
    return pl.pallas_call(
        _mid_body,
        grid=grid,
        in_specs=[
            pl.BlockSpec((_BLK, R1), lambda i: (i, 0)),
            pl.BlockSpec((_BLK, R1), lambda i: (i, 0)),
            pl.BlockSpec((_BLK, R1), lambda i: (i, 0)),
            pl.BlockSpec((_BLK, 16), lambda i: (i, 0)),
            pl.BlockSpec((1, HC1), lambda i: (0, 0)),
            pl.BlockSpec((HC1, D_OUT), lambda i: (0, 0)),
            pl.BlockSpec((16, HC1), lambda i: (0, 0)),
            pl.BlockSpec((D_OUT, R2), lambda i: (0, 0)),
            pl.BlockSpec((D_OUT, 16), lambda i: (0, 0)),
        ],
        out_specs=[
            pl.BlockSpec((_BLK, R2), lambda i: (i, 0)),
            pl.BlockSpec((_BLK, 16), lambda i: (i, 0)),
        ],
        out_shape=[
            jax.ShapeDtypeStruct((N, R2), jnp.float32),
            jax.ShapeDtypeStruct((N, 16), jnp.float32),
        ],
    )(p0, p1, hs1, ad1c, b1, W2, Me, Cat2, Ad2c)


def _fin_body(p0_ref, p1_ref, hs2_ref, adc_ref, b2_ref, m2_ref, o_ref):
    t = p0_ref[...] + p1_ref[...]
    h2 = hs2_ref[:, :D_OUT]
    w = _leaky_exp(hs2_ref[:, D_OUT:D_OUT + 16] + adc_ref[...])
    m2 = m2_ref[...]
    num = t[:, :D_OUT] + jnp.dot(w, m2, preferred_element_type=jnp.float32) * h2
    den = jnp.dot(t[:, D_OUT:D_OUT + 16] + w, m2, preferred_element_type=jnp.float32)
    o_ref[...] = num / (den + 1e-16) + b2_ref[...]


def _fin(p0, p1, hs2, ad2c, b2, M2e):
    R2 = D_OUT + 16
    grid = (N // _BLK,)
    return pl.pallas_call(
        _fin_body,
        grid=grid,
        in_specs=[
            pl.BlockSpec((_BLK, R2), lambda i: (i, 0)),
            pl.BlockSpec((_BLK, R2), lambda i: (i, 0)),
            pl.BlockSpec((_BLK, R2), lambda i: (i, 0)),
            pl.BlockSpec((_BLK, 16), lambda i: (i, 0)),
            pl.BlockSpec((1, D_OUT), lambda i: (0, 0)),
            pl.BlockSpec((16, D_OUT), lambda i: (0, 0)),
        ],
        out_specs=pl.BlockSpec((_BLK, D_OUT), lambda i: (i, 0)),
        out_shape=jax.ShapeDtypeStruct((N, D_OUT), jnp.float32),
    )(p0, p1, hs2, ad2c, b2, M2e)


def _make_edge_kernel(hw: int, k: int):
    r = hw + 16
    nj = hw // 16
    epw = E // 32
    nch = epw // k
    assert nch % 2 == 1 and nch >= 3
    npair = (nch - 1) // 2
    rp = (N // 16) // 8 * 8
    tail = N - 16 * rp
    mesh = plsc.VectorSubcoreMesh(core_axis_name="c", subcore_axis_name="s")

    @functools.partial(
        pl.kernel,
        out_type=jax.ShapeDtypeStruct((2, N, r), jnp.float32),
        mesh=mesh,
        compiler_params=pltpu.CompilerParams(use_tc_tiling_on_sc=False),
        scratch_types=[
            pltpu.VMEM((k,), jnp.int32),
            pltpu.VMEM((k,), jnp.int32),
            pltpu.VMEM((k,), jnp.int32),
            pltpu.VMEM((k,), jnp.int32),
            pltpu.VMEM((k, r), jnp.float32),
            pltpu.VMEM((k, r), jnp.float32),
            pltpu.VMEM((k, 16), jnp.float32),
            pltpu.VMEM((k, 16), jnp.float32),
            pltpu.VMEM((k, r), jnp.float32),
            pltpu.VMEM_SHARED((N, r), jnp.float32),
            pltpu.SemaphoreType.DMA,
            pltpu.SemaphoreType.DMA,
            pltpu.SemaphoreType.DMA,
            pltpu.SemaphoreType.DMA,
            pltpu.SemaphoreType.DMA,
        ],
    )
    def ek(src_h, dst_h, hs_h, ad_h, zz_h, out_h,
           sidx0, sidx1, didx0, didx1, hsb0, hsb1, adb0, adb1, mb,
           acc, gi0, gi1, g0, g1, ss):
        c = lax.axis_index("c")
        s = lax.axis_index("s")
        wid = c * 16 + s
        roff = s * rp
        pltpu.sync_copy(zz_h.at[pl.ds(roff, rp)], acc.at[pl.ds(roff, rp)])

        @pl.when(s == 0)
        def _():
            pltpu.sync_copy(zz_h.at[pl.ds(16 * rp, tail)],
                            acc.at[pl.ds(16 * rp, tail)])

        plsc.subcore_barrier()
        base = wid * epw
        sb = (sidx0, sidx1)
        db = (didx0, didx1)
        hb = (hsb0, hsb1)
        ab = (adb0, adb1)
        gi = (gi0, gi1)
        gg = (g0, g1)

        def sid(i, p):
            pltpu.async_copy(src_h.at[pl.ds(base + i * k, k)], sb[p], gi[p])
            pltpu.async_copy(dst_h.at[pl.ds(base + i * k, k)], db[p], gi[p])

        def wid_(i, p):
            pltpu.make_async_copy(src_h.at[pl.ds(base + i * k, k)], sb[p], gi[p]).wait()
            pltpu.make_async_copy(dst_h.at[pl.ds(base + i * k, k)], db[p], gi[p]).wait()

        def sgath(p):
            pltpu.async_copy(hs_h.at[sb[p]], hb[p], gg[p])
            pltpu.async_copy(ad_h.at[db[p]], ab[p], gg[p])

        def wgath(p):
            pltpu.make_async_copy(hs_h.at[sb[p]], hb[p], gg[p]).wait()
            pltpu.make_async_copy(ad_h.at[db[p]], ab[p], gg[p]).wait()

        def comp(p):
            hsb = hb[p]
            adb = ab[p]

            @functools.partial(plsc.parallel_loop, 0, k, unroll=4)
            def _(e):
                z = hsb[e, pl.ds(hw, 16)] + adb[e, :]
                w = jnp.exp(jnp.where(z >= 0.0, z, z * 0.2))
                mb[e, pl.ds(hw, 16)] = w
                for j in range(nj):
                    mb[e, pl.ds(j * 16, 16)] = (
                        hsb[e, pl.ds(j * 16, 16)] * jnp.full((16,), w[j], jnp.float32))

        def scat(p):
            pltpu.async_copy(mb, acc.at[db[p]], ss, add=True)

        def wscat(p):
            pltpu.make_async_copy(mb, acc.at[db[p]], ss).wait()

        sid(0, 0)
        wid_(0, 0)
        sgath(0)
        sid(1, 1)

        def half(cid, p, first, do_next, do_nextid):
            wgath(p)
            if do_next:
                wid_(cid + 1, 1 - p)
                sgath(1 - p)
            if not first:
                wscat(1 - p)
            comp(p)
            scat(p)
            if do_nextid:
                sid(cid + 2, p)

        def body(j, cc):
            a = 2 * j
            half(a, 0, False, True, True)

            @pl.when(j <= npair - 2)
            def _():
                half(a + 1, 1, False, True, True)

            @pl.when(j == npair - 1)
            def _():
                half(a + 1, 1, False, True, False)

            return cc

        half(0, 0, True, True, True)
        half(1, 1, False, True, True)
        lax.fori_loop(1, npair, body, 0)

        last = nch - 1
        half(last, 0, False, False, False)
        wscat(0)

        plsc.subcore_barrier()
        pltpu.sync_copy(acc.at[pl.ds(roff, rp)], out_h.at[c, pl.ds(roff, rp)])

        @pl.when(s == 0)
        def _():
            pltpu.sync_copy(acc.at[pl.ds(16 * rp, tail)],
                            out_h.at[c, pl.ds(16 * rp, tail)])

    return ek


_edge1 = _make_edge_kernel(HC1, _K)
_edge2 = _make_edge_kernel(D_OUT, _K)


def kernel(x, edge_index, W1, att_src1, att_dst1, bias1,
           W2, att_src2, att_dst2, bias2):
    src = edge_index[0]
    dst = edge_index[1]

    asf1 = att_src1.reshape(HC1)
    adf1 = att_dst1.reshape(HC1)
    i128 = jnp.arange(HC1)
    h16 = jnp.arange(16)
    Mc = ((i128[:, None] // HID) == h16[None, :]).astype(jnp.float32)
    Cat1 = jnp.concatenate([jnp.eye(HC1, dtype=jnp.float32),
                            asf1[:, None] * Mc], axis=1)
    Adst1c = adf1[:, None] * Mc
    Me = (h16[:, None] == (i128[None, :] // HID)).astype(jnp.float32)
    p2 = jnp.concatenate([jnp.ones((1, 4), jnp.float32),
                          jnp.zeros((1, 12), jnp.float32)], axis=1)
    Cat2 = jnp.concatenate([jnp.eye(D_OUT, dtype=jnp.float32),
                            att_src2.reshape(D_OUT, 1) @ p2], axis=1)
    Ad2c = att_dst2.reshape(D_OUT, 1) @ p2
    M2e = jnp.zeros((16, D_OUT), jnp.float32).at[0].set(1.0)

    hs1, ad1c = _lin1(x, W1, Cat1, Adst1c)

    z1 = jnp.zeros((N, HC1 + 16), jnp.float32)
    part1 = _edge1(src, dst, hs1, ad1c, z1)

    hs2, ad2c = _mid(part1[0], part1[1], hs1, ad1c,
                     bias1.reshape(1, HC1), W2, Me, Cat2, Ad2c)

    z2 = jnp.zeros((N, D_OUT + 16), jnp.float32)
    part2 = _edge2(src, dst, hs2, ad2c, z2)

    out = _fin(part2[0], part2[1], hs2, ad2c,
               bias2.reshape(1, D_OUT), M2e)
    return out
